# Initial kernel scaffold; baseline (speedup 1.0000x reference)
import jax
import jax.numpy as jnp
from jax.experimental import pallas as pl


def kernel(x, x_v, node_attr, edge_index, W_embed, W_rad1_0, b_rad1_0, W_rad2_0, W_rad1_1, b_rad1_1, W_rad2_1, W_cg0, W_cg1, W_attr0, W_up0, W_tp1, W_read0, W_read1):
    raise NotImplementedError("write your pallas kernel here")


# speedup vs baseline: 28.4859x; 28.4859x over previous
import functools

import jax
import jax.numpy as jnp
from jax import lax
from jax.experimental import pallas as pl
from jax.experimental.pallas import tpu as pltpu
from jax.experimental.pallas import tpu_sc as plsc

N = 50000
E = 800000
NB = 8
C = 16
SH = 4
RO = 9
HID = 64

EP = 819200
BE = 2048
BN = 2000
SCW = 1024
SCWR = SCW // 8
NWK = 32
EPW_G = EP // NWK
EPW_S = EP // 16
MGW = 512
RPT = N // 16

_SQRT2 = 1.4142135623730951


def _edge_prep_body(x_ref, xv_ref, w1a_ref, b1a_ref, w2a_ref,
                    w1b_ref, b1b_ref, w2b_ref, wtp_ref,
                    sp_ref, r1p_ref, tp_ref):
    rr = x_ref[...] + 1e-6
    nvec = (jnp.arange(1, NB + 1, dtype=jnp.int32)
            .astype(jnp.float32))[None, :]
    rb = _SQRT2 * jnp.sin(nvec * jnp.pi * rr) / rr
    ha = jnp.tanh(jnp.dot(rb, w1a_ref[...], preferred_element_type=jnp.float32)
                  + b1a_ref[...])
    r0 = jnp.dot(ha, w2a_ref[...], preferred_element_type=jnp.float32)
    hb = jnp.tanh(jnp.dot(rb, w1b_ref[...], preferred_element_type=jnp.float32)
                  + b1b_ref[...])
    r1 = jnp.dot(hb, w2b_ref[...], preferred_element_type=jnp.float32)
    t = jnp.dot(xv_ref[...], wtp_ref[...], preferred_element_type=jnp.float32)
    sh = xv_ref[...]
    for i in range(SH):
        sp_ref[i, :, :] = r0 * sh[:, i:i + 1]
    r1p_ref[...] = r1
    tp_ref[...] = t


def _tc_edge_prep(xs, sh, w1a, b1a, w2a, w1b, b1b, w2b, wtp_r):
    full = lambda s: pl.BlockSpec(s, lambda i: (0, 0))
    return pl.pallas_call(
        _edge_prep_body,
        grid=(EP // BE,),
        in_specs=[
            pl.BlockSpec((BE, 1), lambda i: (i, 0)),
            pl.BlockSpec((BE, SH), lambda i: (i, 0)),
            full((NB, HID)), full((1, HID)), full((HID, C)),
            full((NB, HID)), full((1, HID)), full((HID, C)),
            full((SH, 16)),
        ],
        out_specs=[
            pl.BlockSpec((SH, BE, C), lambda i: (0, i, 0)),
            pl.BlockSpec((BE, C), lambda i: (i, 0)),
            pl.BlockSpec((BE, C), lambda i: (i, 0)),
        ],
        out_shape=[
            jax.ShapeDtypeStruct((SH, EP, C), jnp.float32),
            jax.ShapeDtypeStruct((EP, C), jnp.float32),
            jax.ShapeDtypeStruct((EP, C), jnp.float32),
        ],
    )(xs, sh, w1a, b1a, w2a, w1b, b1b, w2b, wtp_r)


def _node_prep_body(na_ref, wemb_ref, wattr_ref, f0_ref, attr_ref):
    na = na_ref[...]
    f0_ref[...] = jnp.dot(na, wemb_ref[...], preferred_element_type=jnp.float32)
    attr_ref[...] = jnp.dot(na, wattr_ref[...],
                            preferred_element_type=jnp.float32)


def _tc_node_prep(na, wemb, wattr):
    full = lambda s: pl.BlockSpec(s, lambda i: (0, 0))
    return pl.pallas_call(
        _node_prep_body,
        grid=(N // BN,),
        in_specs=[pl.BlockSpec((BN, 4), lambda i: (i, 0)),
                  full((4, C)), full((4, C))],
        out_specs=[pl.BlockSpec((BN, C), lambda i: (i, 0)),
                   pl.BlockSpec((BN, C), lambda i: (i, 0))],
        out_shape=[jax.ShapeDtypeStruct((N, C), jnp.float32),
                   jax.ShapeDtypeStruct((N, C), jnp.float32)],
    )(na, wemb, wattr)


def _comb_from_msg(msg_ref, attr_ref, wcg_ref):
    msg = [msg_ref[i, :, :] for i in range(SH)]
    attr = attr_ref[...]
    prods = {}
    for i in range(SH):
        for j in range(i, SH):
            prods[(i, j)] = msg[i] * msg[j]
    comb = []
    for k in range(SH):
        acc = None
        for i in range(SH):
            for j in range(i, SH):
                w = wcg_ref[i, j, k] if i == j else (
                    wcg_ref[i, j, k] + wcg_ref[j, i, k])
                term = w * prods[(i, j)]
                acc = term if acc is None else acc + term
        comb.append(msg[k] + attr * acc)
    return comb


def _node1_body(msg_ref, attr_ref, wcg_ref, wup_ref, f1_ref, rsum_ref):
    comb = _comb_from_msg(msg_ref, attr_ref, wcg_ref)

    @pl.when(pl.program_id(0) == 0)
    def _():
        rsum_ref[...] = jnp.zeros((1, 64), jnp.float32)

    partial = jnp.concatenate(
        [jnp.sum(c, axis=0, keepdims=True) for c in comb], axis=1)
    rsum_ref[...] += partial
    wup = wup_ref[...]
    for i in range(SH):
        f1_ref[i, :, :] = jnp.dot(comb[i], wup,
                                  preferred_element_type=jnp.float32)


def _tc_node1(msg1, attr0, wcg0, wup0):
    return pl.pallas_call(
        _node1_body,
        grid=(N // BN,),
        in_specs=[pl.BlockSpec((SH, BN, C), lambda i: (0, i, 0)),
                  pl.BlockSpec((BN, C), lambda i: (i, 0)),
                  pl.BlockSpec(memory_space=pltpu.SMEM),
                  pl.BlockSpec((C, C), lambda i: (0, 0))],
        out_specs=[pl.BlockSpec((SH, BN, C), lambda i: (0, i, 0)),
                   pl.BlockSpec((1, 64), lambda i: (0, 0))],
        out_shape=[jax.ShapeDtypeStruct((SH, N, C), jnp.float32),
                   jax.ShapeDtypeStruct((1, 64), jnp.float32)],
    )(msg1, attr0, wcg0, wup0)


def _node2_body(msg_ref, attr_ref, wcg_ref, rsum1_ref, wr0_ref, wr1_ref,
                out_ref, acc_ref):
    comb = _comb_from_msg(msg_ref, attr_ref, wcg_ref)

    @pl.when(pl.program_id(0) == 0)
    def _():
        acc_ref[...] = jnp.zeros((1, 64), jnp.float32)

    partial = jnp.concatenate(
        [jnp.sum(c, axis=0, keepdims=True) for c in comb], axis=1)
    acc_ref[...] += partial

    @pl.when(pl.program_id(0) == N // BN - 1)
    def _():
        out_ref[...] = (
            jnp.dot(rsum1_ref[...], wr0_ref[...],
                    preferred_element_type=jnp.float32)
            + jnp.dot(acc_ref[...], wr1_ref[...],
                      preferred_element_type=jnp.float32))


def _tc_node2(msg2, attr0, wcg1, rsum1, wr0t, wr1t):
    return pl.pallas_call(
        _node2_body,
        grid=(N // BN,),
        in_specs=[pl.BlockSpec((SH, BN, C), lambda i: (0, i, 0)),
                  pl.BlockSpec((BN, C), lambda i: (i, 0)),
                  pl.BlockSpec(memory_space=pltpu.SMEM),
                  pl.BlockSpec((1, 64), lambda i: (0, 0)),
                  pl.BlockSpec((64, RO), lambda i: (0, 0)),
                  pl.BlockSpec((64, RO), lambda i: (0, 0))],
        out_specs=pl.BlockSpec((1, RO), lambda i: (0, 0)),
        out_shape=jax.ShapeDtypeStruct((1, RO), jnp.float32),
        scratch_shapes=[pltpu.VMEM((1, 64), jnp.float32)],
    )(msg2, attr0, wcg1, rsum1, wr0t, wr1t)


@functools.cache
def _sc_mesh():
    return plsc.VectorSubcoreMesh(core_axis_name="c", subcore_axis_name="s")


@functools.cache
def _sc_gather_f0():
    @functools.partial(
        pl.kernel,
        out_type=jax.ShapeDtypeStruct((EP, C), jnp.float32),
        mesh=_sc_mesh(),
        compiler_params=pltpu.CompilerParams(use_tc_tiling_on_sc=False),
        scratch_types=[pltpu.VMEM((SCW,), jnp.int32),
                       pltpu.VMEM((SCW, C), jnp.float32),
                       pltpu.SemaphoreType.DMA],
    )
    def k(src_hbm, f0_hbm, out_hbm, idx_v, rows_v, sem):
        cid = lax.axis_index("c")
        sid = lax.axis_index("s")
        base = (sid * 2 + cid) * EPW_G

        def body(w, carry):
            off = base + w * SCW
            pltpu.sync_copy(src_hbm.at[pl.ds(off, SCW)], idx_v)
            pltpu.async_copy(f0_hbm.at[idx_v], rows_v, sem).wait()
            pltpu.sync_copy(rows_v, out_hbm.at[pl.ds(off, SCW)])
            return carry

        lax.fori_loop(0, EPW_G // SCW, body, 0)

    return k


@functools.cache
def _sc_scatter1():
    @functools.partial(
        pl.kernel,
        out_type=jax.ShapeDtypeStruct((SH, N, C), jnp.float32),
        mesh=_sc_mesh(),
        compiler_params=pltpu.CompilerParams(use_tc_tiling_on_sc=False),
        scratch_types=[pltpu.VMEM((SCW,), jnp.int32),
                       pltpu.VMEM((SCW, C), jnp.float32),
                       pltpu.VMEM((SCW, C), jnp.float32),
                       pltpu.VMEM((SCW, C), jnp.float32),
                       pltpu.VMEM_SHARED((N, C), jnp.float32),
                       pltpu.SemaphoreType.DMA],
    )
    def k(dst_hbm, f0g_hbm, sp_hbm, zeros_hbm, out_hbm,
          idx_v, f0g_v, s_v, m_v, acc_sh, sem):
        cid = lax.axis_index("c")
        sid = lax.axis_index("s")
        rbase = sid * RPT
        ebase = sid * EPW_S
        for p in range(2):
            q = 2 * cid + p
            pltpu.sync_copy(zeros_hbm.at[pl.ds(rbase, RPT)],
                            acc_sh.at[pl.ds(rbase, RPT)])
            plsc.subcore_barrier()

            def body(w, carry):
                off = ebase + w * SCW
                pltpu.sync_copy(dst_hbm.at[pl.ds(off, SCW)], idx_v)
                pltpu.sync_copy(f0g_hbm.at[pl.ds(off, SCW)], f0g_v)
                pltpu.sync_copy(sp_hbm.at[q, pl.ds(off, SCW)], s_v)

                def rows(r, c2):
                    for j in range(4):
                        e = 4 * r + j
                        m_v[e, :] = f0g_v[e, :] * s_v[e, :]
                    return c2

                lax.fori_loop(0, SCW // 4, rows, 0)
                pltpu.sync_copy(m_v, acc_sh.at[idx_v], add=True)
                return carry

            lax.fori_loop(0, EPW_S // SCW, body, 0)
            plsc.subcore_barrier()
            pltpu.sync_copy(acc_sh.at[pl.ds(rbase, RPT)],
                            out_hbm.at[q, pl.ds(rbase, RPT)])
            plsc.subcore_barrier()

    return k


@functools.cache
def _sc_mgather2():
    @functools.partial(
        pl.kernel,
        out_type=jax.ShapeDtypeStruct((SH, EP, C), jnp.float32),
        mesh=_sc_mesh(),
        compiler_params=pltpu.CompilerParams(use_tc_tiling_on_sc=False),
        scratch_types=[pltpu.VMEM((MGW,), jnp.int32),
                       [pltpu.VMEM((MGW, C), jnp.float32) for _ in range(SH)],
                       pltpu.VMEM((MGW, C), jnp.float32),
                       pltpu.VMEM((MGW, C), jnp.float32),
                       [pltpu.VMEM((MGW, C), jnp.float32) for _ in range(SH)],
                       pltpu.SemaphoreType.DMA],
    )
    def k(src_hbm, f1_hbm, r1p_hbm, tp_hbm, out_hbm,
          idx_v, f1g_v, r1_v, t_v, m_v, sem):
        cid = lax.axis_index("c")
        sid = lax.axis_index("s")
        base = (sid * 2 + cid) * EPW_G

        def body(w, carry):
            off = base + w * MGW
            pltpu.sync_copy(src_hbm.at[pl.ds(off, MGW)], idx_v)
            for i in range(SH):
                pltpu.async_copy(f1_hbm.at[i].at[idx_v], f1g_v[i], sem).wait()
            pltpu.sync_copy(r1p_hbm.at[pl.ds(off, MGW)], r1_v)
            pltpu.sync_copy(tp_hbm.at[pl.ds(off, MGW)], t_v)

            cidx = [jnp.full((16,), m, jnp.int32) for m in range(16)]

            def rows(r, c2):
                for j in range(4):
                    e = 4 * r + j
                    r1row = r1_v[e, :]
                    trow = t_v[e, :]
                    a = [f1g_v[i][e, :] * r1row for i in range(SH)]
                    for k2 in range(SH):
                        acc = None
                        for i in range(SH):
                            tb = trow[cidx[4 * i + k2]]
                            term = a[i] * tb
                            acc = term if acc is None else acc + term
                        m_v[k2][e, :] = acc
                return c2

            lax.fori_loop(0, MGW // 4, rows, 0)
            for k2 in range(SH):
                pltpu.sync_copy(m_v[k2], out_hbm.at[k2, pl.ds(off, MGW)])
            return carry

        lax.fori_loop(0, EPW_G // MGW, body, 0)

    return k


@functools.cache
def _sc_scatter2():
    @functools.partial(
        pl.kernel,
        out_type=jax.ShapeDtypeStruct((SH, N, C), jnp.float32),
        mesh=_sc_mesh(),
        compiler_params=pltpu.CompilerParams(use_tc_tiling_on_sc=False),
        scratch_types=[pltpu.VMEM((SCW,), jnp.int32),
                       pltpu.VMEM((SCW, C), jnp.float32),
                       pltpu.VMEM_SHARED((N, C), jnp.float32),
                       pltpu.SemaphoreType.DMA],
    )
    def k(dst_hbm, upd_hbm, zeros_hbm, out_hbm, idx_v, upd_v, acc_sh, sem):
        cid = lax.axis_index("c")
        sid = lax.axis_index("s")
        rbase = sid * RPT
        ebase = sid * EPW_S
        for p in range(2):
            q = 2 * cid + p
            pltpu.sync_copy(zeros_hbm.at[pl.ds(rbase, RPT)],
                            acc_sh.at[pl.ds(rbase, RPT)])
            plsc.subcore_barrier()

            def body(w, carry):
                off = ebase + w * SCW
                pltpu.sync_copy(dst_hbm.at[pl.ds(off, SCW)], idx_v)
                pltpu.sync_copy(upd_hbm.at[q, pl.ds(off, SCW)], upd_v)
                pltpu.sync_copy(upd_v, acc_sh.at[idx_v], add=True)
                return carry

            lax.fori_loop(0, EPW_S // SCW, body, 0)
            plsc.subcore_barrier()
            pltpu.sync_copy(acc_sh.at[pl.ds(rbase, RPT)],
                            out_hbm.at[q, pl.ds(rbase, RPT)])
            plsc.subcore_barrier()

    return k


def kernel(x, x_v, node_attr, edge_index, W_embed, W_rad1_0, b_rad1_0,
           W_rad2_0, W_rad1_1, b_rad1_1, W_rad2_1, W_cg0, W_cg1, W_attr0,
           W_up0, W_tp1, W_read0, W_read1):
    pad = EP - E
    xs = jnp.concatenate([x[0], jnp.zeros((pad,), jnp.float32)])[:, None]
    sh = jnp.concatenate([x_v[0], jnp.zeros((pad, SH), jnp.float32)], axis=0)
    na = node_attr[0]
    fill = (jnp.arange(pad, dtype=jnp.int32) % N)
    src = jnp.concatenate([edge_index[0, 0].astype(jnp.int32), fill])
    dst = jnp.concatenate([edge_index[0, 1].astype(jnp.int32), fill])

    wtp_r = jnp.transpose(W_tp1, (1, 0, 2)).reshape(SH, SH * SH)
    wr0t = jnp.transpose(W_read0, (1, 0, 2)).reshape(SH * C, RO)
    wr1t = jnp.transpose(W_read1, (1, 0, 2)).reshape(SH * C, RO)
    b1a = b_rad1_0[None, :]
    b1b = b_rad1_1[None, :]
    zeros = jnp.zeros((N, C), jnp.float32)

    sp, r1p, tp = _tc_edge_prep(xs, sh, W_rad1_0, b1a, W_rad2_0,
                                W_rad1_1, b1b, W_rad2_1, wtp_r)
    f0, attr0 = _tc_node_prep(na, W_embed, W_attr0)
    f0g = _sc_gather_f0()(src, f0)
    msg1 = _sc_scatter1()(dst, f0g, sp, zeros)
    f1, rsum1 = _tc_node1(msg1, attr0, W_cg0, W_up0)
    m2 = _sc_mgather2()(src, f1, r1p, tp)
    msg2 = _sc_scatter2()(dst, m2, zeros)
    out = _tc_node2(msg2, attr0, W_cg1, rsum1, wr0t, wr1t)
    return out


# speedup vs baseline: 39.9018x; 1.4008x over previous
import functools

import jax
import jax.numpy as jnp
from jax import lax
from jax.experimental import pallas as pl
from jax.experimental.pallas import tpu as pltpu
from jax.experimental.pallas import tpu_sc as plsc

N = 50000
E = 800000
NB = 8
C = 16
SH = 4
RO = 9
HID = 64

EP = 819200
BE = 2048
BN = 2000
SCW = 1024
SCWR = SCW // 8
NWK = 32
EPW_G = EP // NWK
EPW_S = EP // 16
MGW = 512
RPT = N // 16

_SQRT2 = 1.4142135623730951


def _edge_prep_body(xp_ref, xvp_ref, sh0_ref, sh1_ref, sh2_ref, sh3_ref,
                    rep8_ref, nvec_ref, rep16_ref,
                    w1a_ref, b1a_ref, w2a_ref, w1b_ref, b1b_ref, w2b_ref,
                    wtp_ref, sp_ref, r1p_ref, tp_ref):
    dot = lambda a, b: jnp.dot(a, b, preferred_element_type=jnp.float32)
    rr = dot(xp_ref[...] + 1e-6, rep8_ref[...])
    rb = _SQRT2 * jnp.sin(nvec_ref[...] * jnp.pi * rr) / rr
    ha = jnp.tanh(dot(rb, w1a_ref[...]) + b1a_ref[...])
    r0p = dot(ha, w2a_ref[...])
    hb = jnp.tanh(dot(rb, w1b_ref[...]) + b1b_ref[...])
    r1p_ref[...] = dot(hb, w2b_ref[...])
    tp_ref[...] = dot(xvp_ref[...], wtp_ref[...])
    rep16 = rep16_ref[...]
    shq = [sh0_ref, sh1_ref, sh2_ref, sh3_ref]
    for i in range(SH):
        sp_ref[i, :, :] = r0p * dot(shq[i][...], rep16)


BR = BE // 8


def _tc_edge_prep(xp, xvp, sh0, sh1, sh2, sh3, rep8, nvec, rep16,
                  w1ab, b1ab, w2ab, w1bb, b1bb, w2bb, wtpb):
    full = lambda s2: pl.BlockSpec(s2, lambda i: (0, 0))
    eb = lambda m: pl.BlockSpec((BR, m), lambda i: (i, 0))
    return pl.pallas_call(
        _edge_prep_body,
        grid=(EP // BE,),
        in_specs=[
            eb(8), eb(32), eb(8), eb(8), eb(8), eb(8),
            full((8, 64)), full((1, 64)), full((8, 128)),
            full((64, 8 * HID)), full((1, 8 * HID)), full((8 * HID, 128)),
            full((64, 8 * HID)), full((1, 8 * HID)), full((8 * HID, 128)),
            full((32, 128)),
        ],
        out_specs=[
            pl.BlockSpec((SH, BR, 128), lambda i: (0, i, 0)),
            pl.BlockSpec((BR, 128), lambda i: (i, 0)),
            pl.BlockSpec((BR, 128), lambda i: (i, 0)),
        ],
        out_shape=[
            jax.ShapeDtypeStruct((SH, EP // 8, 128), jnp.float32),
            jax.ShapeDtypeStruct((EP // 8, 128), jnp.float32),
            jax.ShapeDtypeStruct((EP // 8, 128), jnp.float32),
        ],
    )(xp, xvp, sh0, sh1, sh2, sh3, rep8, nvec, rep16,
      w1ab, b1ab, w2ab, w1bb, b1bb, w2bb, wtpb)


def _node_prep_body(na_ref, wemb_ref, wattr_ref, f0_ref, attr_ref):
    na = na_ref[...]
    f0_ref[...] = jnp.dot(na, wemb_ref[...], preferred_element_type=jnp.float32)
    attr_ref[...] = jnp.dot(na, wattr_ref[...],
                            preferred_element_type=jnp.float32)


def _tc_node_prep(na, wemb, wattr):
    full = lambda s: pl.BlockSpec(s, lambda i: (0, 0))
    return pl.pallas_call(
        _node_prep_body,
        grid=(N // BN,),
        in_specs=[pl.BlockSpec((BN, 4), lambda i: (i, 0)),
                  full((4, C)), full((4, C))],
        out_specs=[pl.BlockSpec((BN, C), lambda i: (i, 0)),
                   pl.BlockSpec((BN, C), lambda i: (i, 0))],
        out_shape=[jax.ShapeDtypeStruct((N, C), jnp.float32),
                   jax.ShapeDtypeStruct((N, C), jnp.float32)],
    )(na, wemb, wattr)


def _comb_from_msg(msg_ref, attr_ref, wcg_ref):
    msg = [msg_ref[i, :, :] for i in range(SH)]
    attr = attr_ref[...]
    prods = {}
    for i in range(SH):
        for j in range(i, SH):
            prods[(i, j)] = msg[i] * msg[j]
    comb = []
    for k in range(SH):
        acc = None
        for i in range(SH):
            for j in range(i, SH):
                w = wcg_ref[i, j, k] if i == j else (
                    wcg_ref[i, j, k] + wcg_ref[j, i, k])
                term = w * prods[(i, j)]
                acc = term if acc is None else acc + term
        comb.append(msg[k] + attr * acc)
    return comb


def _node1_body(msg_ref, attr_ref, wcg_ref, wup_ref, f1_ref, rsum_ref):
    comb = _comb_from_msg(msg_ref, attr_ref, wcg_ref)

    @pl.when(pl.program_id(0) == 0)
    def _():
        rsum_ref[...] = jnp.zeros((1, 64), jnp.float32)

    partial = jnp.concatenate(
        [jnp.sum(c, axis=0, keepdims=True) for c in comb], axis=1)
    rsum_ref[...] += partial
    wup = wup_ref[...]
    for i in range(SH):
        f1_ref[i, :, :] = jnp.dot(comb[i], wup,
                                  preferred_element_type=jnp.float32)


def _tc_node1(msg1, attr0, wcg0, wup0):
    return pl.pallas_call(
        _node1_body,
        grid=(N // BN,),
        in_specs=[pl.BlockSpec((SH, BN, C), lambda i: (0, i, 0)),
                  pl.BlockSpec((BN, C), lambda i: (i, 0)),
                  pl.BlockSpec(memory_space=pltpu.SMEM),
                  pl.BlockSpec((C, C), lambda i: (0, 0))],
        out_specs=[pl.BlockSpec((SH, BN, C), lambda i: (0, i, 0)),
                   pl.BlockSpec((1, 64), lambda i: (0, 0))],
        out_shape=[jax.ShapeDtypeStruct((SH, N, C), jnp.float32),
                   jax.ShapeDtypeStruct((1, 64), jnp.float32)],
    )(msg1, attr0, wcg0, wup0)


def _node2_body(msg_ref, attr_ref, wcg_ref, rsum1_ref, wr0_ref, wr1_ref,
                out_ref, acc_ref):
    comb = _comb_from_msg(msg_ref, attr_ref, wcg_ref)

    @pl.when(pl.program_id(0) == 0)
    def _():
        acc_ref[...] = jnp.zeros((1, 64), jnp.float32)

    partial = jnp.concatenate(
        [jnp.sum(c, axis=0, keepdims=True) for c in comb], axis=1)
    acc_ref[...] += partial

    @pl.when(pl.program_id(0) == N // BN - 1)
    def _():
        out_ref[...] = (
            jnp.dot(rsum1_ref[...], wr0_ref[...],
                    preferred_element_type=jnp.float32)
            + jnp.dot(acc_ref[...], wr1_ref[...],
                      preferred_element_type=jnp.float32))


def _tc_node2(msg2, attr0, wcg1, rsum1, wr0t, wr1t):
    return pl.pallas_call(
        _node2_body,
        grid=(N // BN,),
        in_specs=[pl.BlockSpec((SH, BN, C), lambda i: (0, i, 0)),
                  pl.BlockSpec((BN, C), lambda i: (i, 0)),
                  pl.BlockSpec(memory_space=pltpu.SMEM),
                  pl.BlockSpec((1, 64), lambda i: (0, 0)),
                  pl.BlockSpec((64, RO), lambda i: (0, 0)),
                  pl.BlockSpec((64, RO), lambda i: (0, 0))],
        out_specs=pl.BlockSpec((1, RO), lambda i: (0, 0)),
        out_shape=jax.ShapeDtypeStruct((1, RO), jnp.float32),
        scratch_shapes=[pltpu.VMEM((1, 64), jnp.float32)],
    )(msg2, attr0, wcg1, rsum1, wr0t, wr1t)


@functools.cache
def _sc_mesh():
    return plsc.VectorSubcoreMesh(core_axis_name="c", subcore_axis_name="s")


@functools.cache
def _sc_gather_f0():
    @functools.partial(
        pl.kernel,
        out_type=jax.ShapeDtypeStruct((EP, C), jnp.float32),
        mesh=_sc_mesh(),
        compiler_params=pltpu.CompilerParams(use_tc_tiling_on_sc=False),
        scratch_types=[pltpu.VMEM((SCW,), jnp.int32),
                       pltpu.VMEM((SCW, C), jnp.float32),
                       pltpu.SemaphoreType.DMA],
    )
    def k(src_hbm, f0_hbm, out_hbm, idx_v, rows_v, sem):
        cid = lax.axis_index("c")
        sid = lax.axis_index("s")
        base = (sid * 2 + cid) * EPW_G

        def body(w, carry):
            off = base + w * SCW
            pltpu.sync_copy(src_hbm.at[pl.ds(off, SCW)], idx_v)
            pltpu.async_copy(f0_hbm.at[idx_v], rows_v, sem).wait()
            pltpu.sync_copy(rows_v, out_hbm.at[pl.ds(off, SCW)])
            return carry

        lax.fori_loop(0, EPW_G // SCW, body, 0)

    return k


@functools.cache
def _sc_scatter1():
    @functools.partial(
        pl.kernel,
        out_type=jax.ShapeDtypeStruct((SH, N, C), jnp.float32),
        mesh=_sc_mesh(),
        compiler_params=pltpu.CompilerParams(use_tc_tiling_on_sc=False),
        scratch_types=[pltpu.VMEM((SCW,), jnp.int32),
                       pltpu.VMEM((SCW, C), jnp.float32),
                       pltpu.VMEM((SCW // 8, 128), jnp.float32),
                       pltpu.VMEM((SCW, C), jnp.float32),
                       pltpu.VMEM_SHARED((N, C), jnp.float32),
                       pltpu.SemaphoreType.DMA],
    )
    def k(dst_hbm, f0g_hbm, sp_hbm, zeros_hbm, out_hbm,
          idx_v, f0g_v, s_v, m_v, acc_sh, sem):
        cid = lax.axis_index("c")
        sid = lax.axis_index("s")
        rbase = sid * RPT
        ebase = sid * EPW_S
        for p in range(2):
            q = 2 * cid + p
            pltpu.sync_copy(zeros_hbm.at[pl.ds(rbase, RPT)],
                            acc_sh.at[pl.ds(rbase, RPT)])
            plsc.subcore_barrier()

            def body(w, carry):
                off = ebase + w * SCW
                pltpu.sync_copy(dst_hbm.at[pl.ds(off, SCW)], idx_v)
                pltpu.sync_copy(f0g_hbm.at[pl.ds(off, SCW)], f0g_v)
                pltpu.sync_copy(sp_hbm.at[q, pl.ds(off // 8, SCW // 8)], s_v)

                def rows(r, c2):
                    for j in range(8):
                        e = 8 * r + j
                        m_v[e, :] = f0g_v[e, :] * s_v[r, pl.ds(16 * j, 16)]
                    return c2

                lax.fori_loop(0, SCW // 8, rows, 0)
                pltpu.sync_copy(m_v, acc_sh.at[idx_v], add=True)
                return carry

            lax.fori_loop(0, EPW_S // SCW, body, 0)
            plsc.subcore_barrier()
            pltpu.sync_copy(acc_sh.at[pl.ds(rbase, RPT)],
                            out_hbm.at[q, pl.ds(rbase, RPT)])
            plsc.subcore_barrier()

    return k


@functools.cache
def _sc_mgather2():
    @functools.partial(
        pl.kernel,
        out_type=jax.ShapeDtypeStruct((SH, EP, C), jnp.float32),
        mesh=_sc_mesh(),
        compiler_params=pltpu.CompilerParams(use_tc_tiling_on_sc=False),
        scratch_types=[pltpu.VMEM((MGW,), jnp.int32),
                       [pltpu.VMEM((MGW, C), jnp.float32) for _ in range(SH)],
                       pltpu.VMEM((MGW // 8, 128), jnp.float32),
                       pltpu.VMEM((MGW // 8, 128), jnp.float32),
                       [pltpu.VMEM((MGW, C), jnp.float32) for _ in range(SH)],
                       pltpu.SemaphoreType.DMA],
    )
    def k(src_hbm, f1_hbm, r1p_hbm, tp_hbm, out_hbm,
          idx_v, f1g_v, r1_v, t_v, m_v, sem):
        cid = lax.axis_index("c")
        sid = lax.axis_index("s")
        base = (sid * 2 + cid) * EPW_G

        def body(w, carry):
            off = base + w * MGW
            pltpu.sync_copy(src_hbm.at[pl.ds(off, MGW)], idx_v)
            for i in range(SH):
                pltpu.async_copy(f1_hbm.at[i].at[idx_v], f1g_v[i], sem).wait()
            pltpu.sync_copy(r1p_hbm.at[pl.ds(off // 8, MGW // 8)], r1_v)
            pltpu.sync_copy(tp_hbm.at[pl.ds(off // 8, MGW // 8)], t_v)

            cidx = [jnp.full((16,), m, jnp.int32) for m in range(16)]

            def rows(r, c2):
                for j in range(8):
                    e = 8 * r + j
                    r1row = r1_v[r, pl.ds(16 * j, 16)]
                    trow = t_v[r, pl.ds(16 * j, 16)]
                    a = [f1g_v[i][e, :] * r1row for i in range(SH)]
                    for k2 in range(SH):
                        acc = None
                        for i in range(SH):
                            tb = trow[cidx[4 * i + k2]]
                            term = a[i] * tb
                            acc = term if acc is None else acc + term
                        m_v[k2][e, :] = acc
                return c2

            lax.fori_loop(0, MGW // 8, rows, 0)
            for k2 in range(SH):
                pltpu.sync_copy(m_v[k2], out_hbm.at[k2, pl.ds(off, MGW)])
            return carry

        lax.fori_loop(0, EPW_G // MGW, body, 0)

    return k


@functools.cache
def _sc_scatter2():
    @functools.partial(
        pl.kernel,
        out_type=jax.ShapeDtypeStruct((SH, N, C), jnp.float32),
        mesh=_sc_mesh(),
        compiler_params=pltpu.CompilerParams(use_tc_tiling_on_sc=False),
        scratch_types=[pltpu.VMEM((SCW,), jnp.int32),
                       pltpu.VMEM((SCW, C), jnp.float32),
                       pltpu.VMEM_SHARED((N, C), jnp.float32),
                       pltpu.SemaphoreType.DMA],
    )
    def k(dst_hbm, upd_hbm, zeros_hbm, out_hbm, idx_v, upd_v, acc_sh, sem):
        cid = lax.axis_index("c")
        sid = lax.axis_index("s")
        rbase = sid * RPT
        ebase = sid * EPW_S
        for p in range(2):
            q = 2 * cid + p
            pltpu.sync_copy(zeros_hbm.at[pl.ds(rbase, RPT)],
                            acc_sh.at[pl.ds(rbase, RPT)])
            plsc.subcore_barrier()

            def body(w, carry):
                off = ebase + w * SCW
                pltpu.sync_copy(dst_hbm.at[pl.ds(off, SCW)], idx_v)
                pltpu.sync_copy(upd_hbm.at[q, pl.ds(off, SCW)], upd_v)
                pltpu.sync_copy(upd_v, acc_sh.at[idx_v], add=True)
                return carry

            lax.fori_loop(0, EPW_S // SCW, body, 0)
            plsc.subcore_barrier()
            pltpu.sync_copy(acc_sh.at[pl.ds(rbase, RPT)],
                            out_hbm.at[q, pl.ds(rbase, RPT)])
            plsc.subcore_barrier()

    return k


def kernel(x, x_v, node_attr, edge_index, W_embed, W_rad1_0, b_rad1_0,
           W_rad2_0, W_rad1_1, b_rad1_1, W_rad2_1, W_cg0, W_cg1, W_attr0,
           W_up0, W_tp1, W_read0, W_read1):
    pad = EP - E
    xpad = jnp.concatenate([x[0], jnp.zeros((pad,), jnp.float32)])
    sh = jnp.concatenate([x_v[0], jnp.zeros((pad, SH), jnp.float32)], axis=0)
    xp = xpad.reshape(EP // 8, 8)
    xvp = sh.reshape(EP // 8, 32)
    shq = [sh[:, i].reshape(EP // 8, 8) for i in range(SH)]
    na = node_attr[0]
    fill = (jnp.arange(pad, dtype=jnp.int32) % N)
    src = jnp.concatenate([edge_index[0, 0].astype(jnp.int32), fill])
    dst = jnp.concatenate([edge_index[0, 1].astype(jnp.int32), fill])

    wtp_r = jnp.transpose(W_tp1, (1, 0, 2)).reshape(SH, SH * SH)
    eye8 = jnp.eye(8, dtype=jnp.float32)
    rep8 = jnp.kron(eye8, jnp.ones((1, 8), jnp.float32))
    nvec = jnp.tile(jnp.arange(1, 9, dtype=jnp.float32), 8)[None, :]
    rep16 = jnp.kron(eye8, jnp.ones((1, 16), jnp.float32))
    w1ab = jnp.kron(eye8, W_rad1_0)
    b1ab = jnp.tile(b_rad1_0, 8)[None, :]
    w2ab = jnp.kron(eye8, W_rad2_0)
    w1bb = jnp.kron(eye8, W_rad1_1)
    b1bb = jnp.tile(b_rad1_1, 8)[None, :]
    w2bb = jnp.kron(eye8, W_rad2_1)
    wtpb = jnp.kron(eye8, wtp_r)
    wr0t = jnp.transpose(W_read0, (1, 0, 2)).reshape(SH * C, RO)
    wr1t = jnp.transpose(W_read1, (1, 0, 2)).reshape(SH * C, RO)
    zeros = jnp.zeros((N, C), jnp.float32)

    sp, r1p, tp = _tc_edge_prep(xp, xvp, shq[0], shq[1], shq[2], shq[3],
                                rep8, nvec, rep16,
                                w1ab, b1ab, w2ab, w1bb, b1bb, w2bb, wtpb)
    f0, attr0 = _tc_node_prep(na, W_embed, W_attr0)
    f0g = _sc_gather_f0()(src, f0)
    msg1 = _sc_scatter1()(dst, f0g, sp, zeros)
    f1, rsum1 = _tc_node1(msg1, attr0, W_cg0, W_up0)
    m2 = _sc_mgather2()(src, f1, r1p, tp)
    msg2 = _sc_scatter2()(dst, m2, zeros)
    out = _tc_node2(msg2, attr0, W_cg1, rsum1, wr0t, wr1t)
    return out


# speedup vs baseline: 51.6552x; 1.2946x over previous
import functools

import numpy as _np

import jax
import jax.numpy as jnp
from jax import lax
from jax.experimental import pallas as pl
from jax.experimental.pallas import tpu as pltpu
from jax.experimental.pallas import tpu_sc as plsc

N = 50000
E = 800000
NB = 8
C = 16
SH = 4
RO = 9
HID = 64

EP = 819200
BE = 2048
BN = 2000
SCW = 1024
SCWR = SCW // 8
NWK = 32
EPW_G = EP // NWK
EPW_S = EP // 16
MGW = 512
RPT = N // 16

_SQRT2 = 1.4142135623730951

_SEL = _np.zeros((SH, 32, 128), _np.float32)
for _q in range(SH):
    for _j in range(8):
        _SEL[_q, 4 * _j + _q, 16 * _j:16 * _j + 16] = 1.0
_SLT = _np.zeros((SH, 64, 64), _np.float32)
for _i in range(SH):
    for _k in range(SH):
        _SLT[_i, 16 * _i:16 * _i + 16, 16 * _k:16 * _k + 16] = _np.eye(
            C, dtype=_np.float32)
_TILE4 = _np.tile(_np.eye(C, dtype=_np.float32), (1, SH))


def _edge_prep_body(xp_ref, xvp_ref, rep8_ref, nvec_ref, sel_ref,
                    w1a_ref, b1a_ref, w2a_ref, w1b_ref, b1b_ref, w2b_ref,
                    wtp_ref, sp_ref, r1p_ref, tp_ref):
    dot = lambda a, b: jnp.dot(a, b, preferred_element_type=jnp.float32)
    doth = lambda a, b: jnp.dot(a, b, preferred_element_type=jnp.float32,
                                precision=lax.Precision.HIGHEST)
    rr = doth(xp_ref[...] + 1e-6, rep8_ref[...])
    rb = _SQRT2 * jnp.sin(nvec_ref[...] * jnp.pi * rr) / rr
    ha = jnp.tanh(dot(rb, w1a_ref[...]) + b1a_ref[...])
    r0p = dot(ha, w2a_ref[...])
    hb = jnp.tanh(dot(rb, w1b_ref[...]) + b1b_ref[...])
    r1p_ref[...] = dot(hb, w2b_ref[...])
    tp_ref[...] = dot(xvp_ref[...], wtp_ref[...])
    xvp = xvp_ref[...]
    for i in range(SH):
        sp_ref[i, :, :] = r0p * doth(xvp, sel_ref[i])


BR = BE // 8


def _tc_edge_prep(xp, xvp, rep8, nvec, sel,
                  w1ab, b1ab, w2ab, w1bb, b1bb, w2bb, wtpb):
    full = lambda s2: pl.BlockSpec(s2, lambda i: (0, 0))
    eb = lambda m: pl.BlockSpec((BR, m), lambda i: (i, 0))
    return pl.pallas_call(
        _edge_prep_body,
        grid=(EP // BE,),
        in_specs=[
            eb(8), eb(32),
            full((8, 64)), full((1, 64)),
            pl.BlockSpec((SH, 32, 128), lambda i: (0, 0, 0)),
            full((64, 8 * HID)), full((1, 8 * HID)), full((8 * HID, 128)),
            full((64, 8 * HID)), full((1, 8 * HID)), full((8 * HID, 128)),
            full((32, 128)),
        ],
        out_specs=[
            pl.BlockSpec((SH, BR, 128), lambda i: (0, i, 0)),
            pl.BlockSpec((BR, 128), lambda i: (i, 0)),
            pl.BlockSpec((BR, 128), lambda i: (i, 0)),
        ],
        out_shape=[
            jax.ShapeDtypeStruct((SH, EP // 8, 128), jnp.float32),
            jax.ShapeDtypeStruct((EP // 8, 128), jnp.float32),
            jax.ShapeDtypeStruct((EP // 8, 128), jnp.float32),
        ],
    )(xp, xvp, rep8, nvec, sel,
      w1ab, b1ab, w2ab, w1bb, b1bb, w2bb, wtpb)


def _node_prep_body(na_ref, wemb_ref, wattr_ref, f0_ref, attr_ref):
    na = na_ref[...]
    f0_ref[...] = jnp.dot(na, wemb_ref[...], preferred_element_type=jnp.float32)
    attr_ref[...] = jnp.dot(na, wattr_ref[...],
                            preferred_element_type=jnp.float32)


def _tc_node_prep(na, wemb, wattr):
    full = lambda s: pl.BlockSpec(s, lambda i: (0, 0))
    return pl.pallas_call(
        _node_prep_body,
        grid=(N // BN,),
        in_specs=[pl.BlockSpec((BN, 4), lambda i: (i, 0)),
                  full((4, C)), full((4, C))],
        out_specs=[pl.BlockSpec((BN, C), lambda i: (i, 0)),
                   pl.BlockSpec((BN, C), lambda i: (i, 0))],
        out_shape=[jax.ShapeDtypeStruct((N, C), jnp.float32),
                   jax.ShapeDtypeStruct((N, C), jnp.float32)],
    )(na, wemb, wattr)


def _comb64(msg_all, attr_ref, wcgv_ref, slt_ref, tile4):
    dot = lambda a, b: jnp.dot(a, b, preferred_element_type=jnp.float32,
                               precision=lax.Precision.HIGHEST)
    m_t = [dot(msg_all, slt_ref[i]) for i in range(SH)]
    prod = None
    idx = 0
    for i in range(SH):
        for j in range(i, SH):
            term = (m_t[i] * m_t[j]) * wcgv_ref[idx:idx + 1, :]
            prod = term if prod is None else prod + term
            idx += 1
    attr_t = dot(attr_ref[...], tile4)
    return msg_all + attr_t * prod


def _node1_body(msg_ref, attr_ref, wcgv_ref, wup4_ref, slt_ref, tile4_ref,
                f1_ref, rsum_ref):
    dot = lambda a, b: jnp.dot(a, b, preferred_element_type=jnp.float32,
                               precision=lax.Precision.HIGHEST)
    comb_all = _comb64(msg_ref[...], attr_ref, wcgv_ref, slt_ref,
                       tile4_ref[...])

    @pl.when(pl.program_id(0) == 0)
    def _():
        rsum_ref[...] = jnp.zeros((1, 64), jnp.float32)

    rsum_ref[...] += jnp.sum(comb_all, axis=0, keepdims=True)
    f1_ref[...] = dot(comb_all, wup4_ref[...])


def _tc_node1(msg1, attr0, wcgv0, wup4):
    return pl.pallas_call(
        _node1_body,
        grid=(N // BN,),
        in_specs=[pl.BlockSpec((BN, 64), lambda i: (i, 0)),
                  pl.BlockSpec((BN, C), lambda i: (i, 0)),
                  pl.BlockSpec((10, 64), lambda i: (0, 0)),
                  pl.BlockSpec((64, 64), lambda i: (0, 0)),
                  pl.BlockSpec((SH, 64, 64), lambda i: (0, 0, 0)),
                  pl.BlockSpec((C, 64), lambda i: (0, 0))],
        out_specs=[pl.BlockSpec((BN, 64), lambda i: (i, 0)),
                   pl.BlockSpec((1, 64), lambda i: (0, 0))],
        out_shape=[jax.ShapeDtypeStruct((N, 64), jnp.float32),
                   jax.ShapeDtypeStruct((1, 64), jnp.float32)],
    )(msg1, attr0, wcgv0, wup4, jnp.asarray(_SLT), jnp.asarray(_TILE4))


def _node2_body(msg_ref, attr_ref, wcgv_ref, rsum1_ref, wr0_ref, wr1_ref,
                slt_ref, tile4_ref, out_ref, acc_ref):
    comb_all = _comb64(msg_ref[...], attr_ref, wcgv_ref, slt_ref,
                       tile4_ref[...])

    @pl.when(pl.program_id(0) == 0)
    def _():
        acc_ref[...] = jnp.zeros((1, 64), jnp.float32)

    acc_ref[...] += jnp.sum(comb_all, axis=0, keepdims=True)

    @pl.when(pl.program_id(0) == N // BN - 1)
    def _():
        out_ref[...] = (
            jnp.dot(rsum1_ref[...], wr0_ref[...],
                    preferred_element_type=jnp.float32,
                    precision=lax.Precision.HIGHEST)
            + jnp.dot(acc_ref[...], wr1_ref[...],
                      preferred_element_type=jnp.float32,
                      precision=lax.Precision.HIGHEST))


def _tc_node2(msg2, attr0, wcgv1, rsum1, wr0t, wr1t):
    return pl.pallas_call(
        _node2_body,
        grid=(N // BN,),
        in_specs=[pl.BlockSpec((BN, 64), lambda i: (i, 0)),
                  pl.BlockSpec((BN, C), lambda i: (i, 0)),
                  pl.BlockSpec((10, 64), lambda i: (0, 0)),
                  pl.BlockSpec((1, 64), lambda i: (0, 0)),
                  pl.BlockSpec((64, RO), lambda i: (0, 0)),
                  pl.BlockSpec((64, RO), lambda i: (0, 0)),
                  pl.BlockSpec((SH, 64, 64), lambda i: (0, 0, 0)),
                  pl.BlockSpec((C, 64), lambda i: (0, 0))],
        out_specs=pl.BlockSpec((1, RO), lambda i: (0, 0)),
        out_shape=jax.ShapeDtypeStruct((1, RO), jnp.float32),
        scratch_shapes=[pltpu.VMEM((1, 64), jnp.float32)],
    )(msg2, attr0, wcgv1, rsum1, wr0t, wr1t,
      jnp.asarray(_SLT), jnp.asarray(_TILE4))


@functools.cache
def _sc_mesh():
    return plsc.VectorSubcoreMesh(core_axis_name="c", subcore_axis_name="s")


@functools.cache
def _sc_gather_f0():
    @functools.partial(
        pl.kernel,
        out_type=jax.ShapeDtypeStruct((EP, C), jnp.float32),
        mesh=_sc_mesh(),
        compiler_params=pltpu.CompilerParams(use_tc_tiling_on_sc=False),
        scratch_types=[pltpu.VMEM((SCW,), jnp.int32),
                       pltpu.VMEM((SCW, C), jnp.float32),
                       pltpu.SemaphoreType.DMA],
    )
    def k(src_hbm, f0_hbm, out_hbm, idx_v, rows_v, sem):
        cid = lax.axis_index("c")
        sid = lax.axis_index("s")
        base = (sid * 2 + cid) * EPW_G

        def body(w, carry):
            off = base + w * SCW
            pltpu.sync_copy(src_hbm.at[pl.ds(off, SCW)], idx_v)
            pltpu.async_copy(f0_hbm.at[idx_v], rows_v, sem).wait()
            pltpu.sync_copy(rows_v, out_hbm.at[pl.ds(off, SCW)])
            return carry

        lax.fori_loop(0, EPW_G // SCW, body, 0)

    return k


@functools.cache
def _sc_scatter1():
    @functools.partial(
        pl.kernel,
        out_type=jax.ShapeDtypeStruct((N, 64), jnp.float32),
        mesh=_sc_mesh(),
        compiler_params=pltpu.CompilerParams(use_tc_tiling_on_sc=False),
        scratch_types=[pltpu.VMEM((SCW,), jnp.int32),
                       pltpu.VMEM((SCW, C), jnp.float32),
                       pltpu.VMEM((SCW // 8, 128), jnp.float32),
                       pltpu.VMEM((SCW, C), jnp.float32),
                       pltpu.VMEM_SHARED((N, C), jnp.float32),
                       pltpu.SemaphoreType.DMA],
    )
    def k(dst_hbm, f0g_hbm, sp_hbm, zeros_hbm, out_hbm,
          idx_v, f0g_v, s_v, m_v, acc_sh, sem):
        cid = lax.axis_index("c")
        sid = lax.axis_index("s")
        rbase = sid * RPT
        ebase = sid * EPW_S
        for p in range(2):
            q = 2 * cid + p
            pltpu.sync_copy(zeros_hbm.at[pl.ds(rbase, RPT)],
                            acc_sh.at[pl.ds(rbase, RPT)])
            plsc.subcore_barrier()

            def body(w, carry):
                off = ebase + w * SCW
                d1 = pltpu.async_copy(dst_hbm.at[pl.ds(off, SCW)], idx_v,
                                      sem)
                d2 = pltpu.async_copy(f0g_hbm.at[pl.ds(off, SCW)], f0g_v, sem)
                d3 = pltpu.async_copy(sp_hbm.at[q, pl.ds(off // 8, SCW // 8)],
                                      s_v, sem)
                d1.wait()
                d2.wait()
                d3.wait()

                def rows(r, c2):
                    for j in range(8):
                        e = 8 * r + j
                        m_v[e, :] = f0g_v[e, :] * s_v[r, pl.ds(16 * j, 16)]
                    return c2

                lax.fori_loop(0, SCW // 8, rows, 0)
                pltpu.sync_copy(m_v, acc_sh.at[idx_v], add=True)
                return carry

            lax.fori_loop(0, EPW_S // SCW, body, 0)
            plsc.subcore_barrier()
            pltpu.sync_copy(acc_sh.at[pl.ds(rbase, RPT)],
                            out_hbm.at[pl.ds(rbase, RPT),
                                       pl.ds(16 * q, 16)])
            plsc.subcore_barrier()

    return k


@functools.cache
def _sc_mgather2():
    @functools.partial(
        pl.kernel,
        out_type=jax.ShapeDtypeStruct((SH, EP, C), jnp.float32),
        mesh=_sc_mesh(),
        compiler_params=pltpu.CompilerParams(use_tc_tiling_on_sc=False),
        scratch_types=[pltpu.VMEM((MGW,), jnp.int32),
                       pltpu.VMEM((MGW, 64), jnp.float32),
                       pltpu.VMEM((MGW // 8, 128), jnp.float32),
                       pltpu.VMEM((MGW // 8, 128), jnp.float32),
                       [pltpu.VMEM((MGW, C), jnp.float32) for _ in range(SH)],
                       pltpu.SemaphoreType.DMA],
    )
    def k(src_hbm, f1_hbm, r1p_hbm, tp_hbm, out_hbm,
          idx_v, f1g_v, r1_v, t_v, m_v, sem):
        cid = lax.axis_index("c")
        sid = lax.axis_index("s")
        base = (sid * 2 + cid) * EPW_G

        def body(w, carry):
            off = base + w * MGW
            pltpu.sync_copy(src_hbm.at[pl.ds(off, MGW)], idx_v)
            d1 = pltpu.async_copy(f1_hbm.at[idx_v], f1g_v, sem)
            d2 = pltpu.async_copy(r1p_hbm.at[pl.ds(off // 8, MGW // 8)],
                                  r1_v, sem)
            d3 = pltpu.async_copy(tp_hbm.at[pl.ds(off // 8, MGW // 8)],
                                  t_v, sem)
            d1.wait()
            d2.wait()
            d3.wait()

            cidx = [jnp.full((16,), m, jnp.int32) for m in range(16)]

            def rows(r, c2):
                for j in range(8):
                    e = 8 * r + j
                    r1row = r1_v[r, pl.ds(16 * j, 16)]
                    trow = t_v[r, pl.ds(16 * j, 16)]
                    a = [f1g_v[e, pl.ds(16 * i, 16)] * r1row
                         for i in range(SH)]
                    for k2 in range(SH):
                        acc = None
                        for i in range(SH):
                            tb = trow[cidx[4 * i + k2]]
                            term = a[i] * tb
                            acc = term if acc is None else acc + term
                        m_v[k2][e, :] = acc
                return c2

            lax.fori_loop(0, MGW // 8, rows, 0)
            outs = [pltpu.async_copy(m_v[k2], out_hbm.at[k2, pl.ds(off, MGW)],
                                     sem) for k2 in range(SH)]
            for d in outs:
                d.wait()
            return carry

        lax.fori_loop(0, EPW_G // MGW, body, 0)

    return k


@functools.cache
def _sc_scatter2():
    @functools.partial(
        pl.kernel,
        out_type=jax.ShapeDtypeStruct((N, 64), jnp.float32),
        mesh=_sc_mesh(),
        compiler_params=pltpu.CompilerParams(use_tc_tiling_on_sc=False),
        scratch_types=[pltpu.VMEM((SCW,), jnp.int32),
                       pltpu.VMEM((SCW, C), jnp.float32),
                       pltpu.VMEM_SHARED((N, C), jnp.float32),
                       pltpu.SemaphoreType.DMA],
    )
    def k(dst_hbm, upd_hbm, zeros_hbm, out_hbm, idx_v, upd_v, acc_sh, sem):
        cid = lax.axis_index("c")
        sid = lax.axis_index("s")
        rbase = sid * RPT
        ebase = sid * EPW_S
        for p in range(2):
            q = 2 * cid + p
            pltpu.sync_copy(zeros_hbm.at[pl.ds(rbase, RPT)],
                            acc_sh.at[pl.ds(rbase, RPT)])
            plsc.subcore_barrier()

            def body(w, carry):
                off = ebase + w * SCW
                d1 = pltpu.async_copy(dst_hbm.at[pl.ds(off, SCW)], idx_v,
                                      sem)
                d2 = pltpu.async_copy(upd_hbm.at[q, pl.ds(off, SCW)], upd_v,
                                      sem)
                d1.wait()
                d2.wait()
                pltpu.sync_copy(upd_v, acc_sh.at[idx_v], add=True)
                return carry

            lax.fori_loop(0, EPW_S // SCW, body, 0)
            plsc.subcore_barrier()
            pltpu.sync_copy(acc_sh.at[pl.ds(rbase, RPT)],
                            out_hbm.at[pl.ds(rbase, RPT),
                                       pl.ds(16 * q, 16)])
            plsc.subcore_barrier()

    return k


def kernel(x, x_v, node_attr, edge_index, W_embed, W_rad1_0, b_rad1_0,
           W_rad2_0, W_rad1_1, b_rad1_1, W_rad2_1, W_cg0, W_cg1, W_attr0,
           W_up0, W_tp1, W_read0, W_read1):
    pad = EP - E
    xpad = jnp.concatenate([x[0], jnp.zeros((pad,), jnp.float32)])
    xp = xpad.reshape(EP // 8, 8)
    xvp = jnp.concatenate(
        [x_v[0].reshape(E // 8, 32),
         jnp.zeros((pad // 8, 32), jnp.float32)], axis=0)
    na = node_attr[0]
    fill = (jnp.arange(pad, dtype=jnp.int32) % N)
    src = jnp.concatenate([edge_index[0, 0].astype(jnp.int32), fill])
    dst = jnp.concatenate([edge_index[0, 1].astype(jnp.int32), fill])

    wtp_r = jnp.transpose(W_tp1, (1, 0, 2)).reshape(SH, SH * SH)
    eye8 = jnp.eye(8, dtype=jnp.float32)
    rep8 = jnp.kron(eye8, jnp.ones((1, 8), jnp.float32))
    nvec = jnp.tile(jnp.arange(1, 9, dtype=jnp.float32), 8)[None, :]
    w1ab = jnp.kron(eye8, W_rad1_0)
    b1ab = jnp.tile(b_rad1_0, 8)[None, :]
    w2ab = jnp.kron(eye8, W_rad2_0)
    w1bb = jnp.kron(eye8, W_rad1_1)
    b1bb = jnp.tile(b_rad1_1, 8)[None, :]
    w2bb = jnp.kron(eye8, W_rad2_1)
    wtpb = jnp.kron(eye8, wtp_r)
    wup4 = jnp.kron(jnp.eye(SH, dtype=jnp.float32), W_up0)
    wcgv0_rows = []
    for i in range(SH):
        for j in range(i, SH):
            w = W_cg0[i, j, :] + (W_cg0[j, i, :] if i < j else 0.0)
            wcgv0_rows.append(jnp.repeat(w, C))
    wcgv0 = jnp.stack(wcgv0_rows)
    wcgv1_rows = []
    for i in range(SH):
        for j in range(i, SH):
            w = W_cg1[i, j, :] + (W_cg1[j, i, :] if i < j else 0.0)
            wcgv1_rows.append(jnp.repeat(w, C))
    wcgv1 = jnp.stack(wcgv1_rows)
    wr0t = jnp.transpose(W_read0, (1, 0, 2)).reshape(SH * C, RO)
    wr1t = jnp.transpose(W_read1, (1, 0, 2)).reshape(SH * C, RO)
    zeros = jnp.zeros((N, C), jnp.float32)

    sp, r1p, tp = _tc_edge_prep(xp, xvp, rep8, nvec, jnp.asarray(_SEL),
                                w1ab, b1ab, w2ab, w1bb, b1bb, w2bb, wtpb)
    f0, attr0 = _tc_node_prep(na, W_embed, W_attr0)
    f0g = _sc_gather_f0()(src, f0)
    msg1 = _sc_scatter1()(dst, f0g, sp, zeros)
    f1, rsum1 = _tc_node1(msg1, attr0, wcgv0, wup4)
    m2 = _sc_mgather2()(src, f1, r1p, tp)
    msg2 = _sc_scatter2()(dst, m2, zeros)
    out = _tc_node2(msg2, attr0, wcgv1, rsum1, wr0t, wr1t)
    return out
